# baseline (speedup 1.0000x reference)
import functools

import jax
import jax.numpy as jnp
from jax import lax
from jax.experimental import pallas as pl
from jax.experimental.pallas import tpu as pltpu
from jax.experimental.pallas import tpu_sc as plsc

N_NODES = 10000
N_EDGES = 320000
D = 128

NC = 2
NS = 16
NW = NC * NS

K_SPLIT = 10
CHUNK_EDGES = N_EDGES // K_SPLIT
EPW = CHUNK_EDGES // NW
SUB = 200
N_SUB = EPW // SUB


def _mm_bias_kernel(h_ref, w_ref, b_ref, t_ref, big_ref):
    del big_ref
    w1 = w_ref[:, :D]
    t_ref[...] = (
        lax.dot_general(
            h_ref[...], w1, (((1,), (1,)), ((), ())),
            preferred_element_type=jnp.float32,
        )
        + b_ref[...]
    )


def _node_transform(H, W, b2d):
    return pl.pallas_call(
        _mm_bias_kernel,
        out_shape=[
            jax.ShapeDtypeStruct((N_NODES, D), jnp.float32),
            jax.ShapeDtypeStruct((N_EDGES, D), jnp.float32),
        ],
        out_specs=[
            pl.BlockSpec((N_NODES, D), lambda: (0, 0)),
            pl.BlockSpec(memory_space=pl.ANY),
        ],
    )(H, W, b2d)


def _make_sc_gather_body(chunk_id):
    def body(
        table_hbm, idx_hbm, out_hbm, idx_all, rows0, rows1, rows2,
        gsem0, gsem1, gsem2, osem0, osem1, osem2,
    ):
        wid = lax.axis_index("s") * NC + lax.axis_index("c")
        gbase = chunk_id * CHUNK_EDGES + wid * EPW
        obase = wid * EPW

        rows = (rows0, rows1, rows2)
        gsems = (gsem0, gsem1, gsem2)
        osems = (osem0, osem1, osem2)

        pltpu.sync_copy(idx_hbm.at[pl.ds(gbase, EPW)], idx_all)

        def g_start(c):
            b = c % 3
            pltpu.async_copy(
                table_hbm.at[idx_all.at[pl.ds(c * SUB, SUB)]], rows[b], gsems[b]
            )

        def g_wait(c):
            b = c % 3
            pltpu.make_async_copy(
                table_hbm.at[idx_all.at[pl.ds(c * SUB, SUB)]], rows[b], gsems[b]
            ).wait()

        def s_start(c):
            b = c % 3
            pltpu.async_copy(
                rows[b], out_hbm.at[pl.ds(obase + c * SUB, SUB)], osems[b]
            )

        def s_wait(c):
            b = c % 3
            pltpu.make_async_copy(
                rows[b], out_hbm.at[pl.ds(obase + c * SUB, SUB)], osems[b]
            ).wait()

        g_start(0)
        g_start(1)
        for c in range(N_SUB):
            g_wait(c)
            s_start(c)
            if c >= 1:
                s_wait(c - 1)
            if c + 2 < N_SUB:
                g_start(c + 2)
        s_wait(N_SUB - 1)

    return body


def _sc_gather_chunk(table, heads, chunk_id):
    mesh = plsc.VectorSubcoreMesh(core_axis_name="c", subcore_axis_name="s")
    k = functools.partial(
        pl.kernel,
        mesh=mesh,
        out_type=jax.ShapeDtypeStruct((CHUNK_EDGES, D), jnp.float32),
        scratch_types=[
            pltpu.VMEM((EPW,), jnp.int32),
            pltpu.VMEM((SUB, D), jnp.float32),
            pltpu.VMEM((SUB, D), jnp.float32),
            pltpu.VMEM((SUB, D), jnp.float32),
            pltpu.SemaphoreType.DMA,
            pltpu.SemaphoreType.DMA,
            pltpu.SemaphoreType.DMA,
            pltpu.SemaphoreType.DMA,
            pltpu.SemaphoreType.DMA,
            pltpu.SemaphoreType.DMA,
        ],
        name=f"sc_gather_chunk{chunk_id}",
    )(_make_sc_gather_body(chunk_id))
    return k(table, heads)


def _add_mm_kernel(big_ref, g_ref, e_ref, w_ref, o_ref):
    del big_ref
    w2 = w_ref[:, D:]
    o_ref[...] = g_ref[...] + lax.dot_general(
        e_ref[...], w2, (((1,), (1,)), ((), ())),
        preferred_element_type=jnp.float32,
    )


def _edge_transform_chunk(big, Gi, E, W, chunk_id, blk):
    nb = CHUNK_EDGES // blk
    cb = chunk_id * nb
    return pl.pallas_call(
        _add_mm_kernel,
        grid=(nb,),
        in_specs=[
            pl.BlockSpec(memory_space=pl.ANY),
            pl.BlockSpec((blk, D), lambda j: (j, 0)),
            pl.BlockSpec((blk, D), lambda j: (cb + j, 0)),
            pl.BlockSpec((D, 2 * D), lambda j: (0, 0)),
        ],
        out_specs=pl.BlockSpec((blk, D), lambda j: (cb + j, 0)),
        out_shape=jax.ShapeDtypeStruct((N_EDGES, D), jnp.float32),
        input_output_aliases={0: 0},
        name=f"edge_transform_chunk{chunk_id}",
    )(big, Gi, E, W)


@jax.jit
def kernel(H, E, heads, queries, W, b):
    b2d = b.reshape(1, D)
    heads32 = heads.astype(jnp.int32)
    T, big = _node_transform(H, W, b2d)
    Gs = [_sc_gather_chunk(T, heads32, i) for i in range(K_SPLIT)]
    out = big
    for i in range(K_SPLIT):
        out = _edge_transform_chunk(out, Gs[i], E, W, i, blk=8000)
    return out


# speedup vs baseline: 1.0329x; 1.0329x over previous
import functools

import jax
import jax.numpy as jnp
from jax import lax
from jax.experimental import pallas as pl
from jax.experimental.pallas import tpu as pltpu
from jax.experimental.pallas import tpu_sc as plsc

N_NODES = 10000
N_EDGES = 320000
D = 128

NC = 2
NS = 16
NW = NC * NS
EDGES_PER_W = N_EDGES // NW
SUB = 200
N_SUB = EDGES_PER_W // SUB
N_TRIPLES = (N_SUB - 2) // 3


def _mm_bias_kernel(h_ref, w_ref, b_ref, t_ref):
    w1 = w_ref[:, :D]
    t_ref[...] = (
        lax.dot_general(
            h_ref[...], w1, (((1,), (1,)), ((), ())),
            preferred_element_type=jnp.float32,
        )
        + b_ref[...]
    )


def _node_transform(H, W, b2d):
    return pl.pallas_call(
        _mm_bias_kernel,
        out_shape=jax.ShapeDtypeStruct((N_NODES, D), jnp.float32),
    )(H, W, b2d)


def _sc_gather_body(
    table_hbm, idx_hbm, out_hbm, idx_all, rows0, rows1, rows2,
    gsem0, gsem1, gsem2, osem0, osem1, osem2,
):
    wid = lax.axis_index("s") * NC + lax.axis_index("c")
    base = wid * EDGES_PER_W

    rows = (rows0, rows1, rows2)
    gsems = (gsem0, gsem1, gsem2)
    osems = (osem0, osem1, osem2)

    pltpu.sync_copy(idx_hbm.at[pl.ds(base, EDGES_PER_W)], idx_all)

    def idx_slice(c):
        return idx_all.at[pl.ds(c * SUB, SUB)]

    def out_slice(c):
        return out_hbm.at[pl.ds(base + c * SUB, SUB)]

    def g_start(c, b):
        pltpu.async_copy(table_hbm.at[idx_slice(c)], rows[b], gsems[b])

    def g_wait(c, b):
        pltpu.make_async_copy(table_hbm.at[idx_slice(c)], rows[b], gsems[b]).wait()

    def s_start(c, b):
        pltpu.async_copy(rows[b], out_slice(c), osems[b])

    def s_wait(c, b):
        pltpu.make_async_copy(rows[b], out_slice(c), osems[b]).wait()

    g_start(0, 0)
    g_start(1, 1)
    g_wait(0, 0)
    s_start(0, 0)
    g_start(2, 2)
    g_wait(1, 1)
    s_start(1, 1)
    s_wait(0, 0)
    g_start(3, 0)

    def triple(j, carry):
        c = 3 * j + 2
        g_wait(c, 2)
        s_start(c, 2)
        s_wait(c - 1, 1)
        g_start(c + 2, 1)
        g_wait(c + 1, 0)
        s_start(c + 1, 0)
        s_wait(c, 2)
        g_start(c + 3, 2)
        g_wait(c + 2, 1)
        s_start(c + 2, 1)
        s_wait(c + 1, 0)
        g_start(c + 4, 0)
        return carry

    lax.fori_loop(0, N_TRIPLES - 1, triple, 0)

    c = N_SUB - 3
    g_wait(c, 2)
    s_start(c, 2)
    s_wait(c - 1, 1)
    g_start(c + 2, 1)
    g_wait(c + 1, 0)
    s_start(c + 1, 0)
    s_wait(c, 2)
    g_wait(c + 2, 1)
    s_start(c + 2, 1)
    s_wait(c + 1, 0)
    s_wait(c + 2, 1)


def _sc_gather(table, heads):
    mesh = plsc.VectorSubcoreMesh(core_axis_name="c", subcore_axis_name="s")
    k = functools.partial(
        pl.kernel,
        mesh=mesh,
        out_type=jax.ShapeDtypeStruct((N_EDGES, D), jnp.float32),
        scratch_types=[
            pltpu.VMEM((EDGES_PER_W,), jnp.int32),
            pltpu.VMEM((SUB, D), jnp.float32),
            pltpu.VMEM((SUB, D), jnp.float32),
            pltpu.VMEM((SUB, D), jnp.float32),
            pltpu.SemaphoreType.DMA,
            pltpu.SemaphoreType.DMA,
            pltpu.SemaphoreType.DMA,
            pltpu.SemaphoreType.DMA,
            pltpu.SemaphoreType.DMA,
            pltpu.SemaphoreType.DMA,
        ],
    )(_sc_gather_body)
    return k(table, heads)


def _add_mm_kernel(g_ref, e_ref, w_ref, o_ref):
    w2 = w_ref[:, D:]
    o_ref[...] = g_ref[...] + lax.dot_general(
        e_ref[...], w2, (((1,), (1,)), ((), ())),
        preferred_element_type=jnp.float32,
    )


def _edge_transform(G, E, W, blk):
    n_blocks = N_EDGES // blk
    return pl.pallas_call(
        _add_mm_kernel,
        grid=(n_blocks,),
        in_specs=[
            pl.BlockSpec((blk, D), lambda i: (i, 0)),
            pl.BlockSpec((blk, D), lambda i: (i, 0)),
            pl.BlockSpec((D, 2 * D), lambda i: (0, 0)),
        ],
        out_specs=pl.BlockSpec((blk, D), lambda i: (i, 0)),
        out_shape=jax.ShapeDtypeStruct((N_EDGES, D), jnp.float32),
    )(G, E, W)


@jax.jit
def kernel(H, E, heads, queries, W, b):
    b2d = b.reshape(1, D)
    T = _node_transform(H, W, b2d)
    G = _sc_gather(T, heads.astype(jnp.int32))
    return _edge_transform(G, E, W, blk=8000)
